# baseline (speedup 1.0000x reference)
import functools

import jax
import jax.numpy as jnp
from jax import lax
from jax.experimental import pallas as pl
from jax.experimental.pallas import tpu as pltpu
from jax.experimental.pallas import tpu_sc as plsc

_N = 256
_NH = 32
_NN = _N * _N
_ROWS = 961
_NW = 32
_CHUNK = _NN // _NW
_L = 16


def _sc_gather(table, idx_flat):
    mesh = plsc.VectorSubcoreMesh(core_axis_name="c", subcore_axis_name="s")

    @functools.partial(
        pl.kernel,
        mesh=mesh,
        compiler_params=pltpu.CompilerParams(needs_layout_passes=False),
        out_type=jax.ShapeDtypeStruct((_NH, _NN), jnp.float32),
        scratch_types=[
            pltpu.VMEM((_ROWS * _NH,), jnp.float32),
            pltpu.VMEM((_CHUNK,), jnp.int32),
            pltpu.VMEM((_NH, _CHUNK), jnp.float32),
        ],
    )
    def body(table_hbm, idx_hbm, out_hbm, tab_v, idx_v, out_v):
        wid = lax.axis_index("s") * 2 + lax.axis_index("c")
        base = wid * _CHUNK
        pltpu.sync_copy(table_hbm, tab_v)
        pltpu.sync_copy(idx_hbm.at[pl.ds(base, _CHUNK)], idx_v)

        @plsc.parallel_loop(0, _CHUNK // _L, unroll=4)
        def step(t):
            off = t * _L
            ivec = idx_v[pl.ds(off, _L)]
            for h in range(_NH):
                out_v[h, pl.ds(off, _L)] = plsc.load_gather(
                    tab_v, [ivec + (h * _ROWS)]
                )
        pltpu.sync_copy(out_v, out_hbm.at[:, pl.ds(base, _CHUNK)])

    return body(table, idx_flat)


def kernel(table, rel_index):
    idx_flat = rel_index[:_N, :_N].reshape(-1)
    out = _sc_gather(table.T.reshape(-1), idx_flat)
    return out.reshape(1, _NH, _N, _N)


# speedup vs baseline: 1.0260x; 1.0260x over previous
import functools

import jax
import jax.numpy as jnp
from jax import lax
from jax.experimental import pallas as pl
from jax.experimental.pallas import tpu as pltpu
from jax.experimental.pallas import tpu_sc as plsc

_N = 256
_NH = 32
_NN = _N * _N
_ROWS = 961
_NW = 32
_CHUNK = _NN // _NW
_L = 16


def _sc_gather(table, idx_flat):
    mesh = plsc.VectorSubcoreMesh(core_axis_name="c", subcore_axis_name="s")

    @functools.partial(
        pl.kernel,
        mesh=mesh,
        compiler_params=pltpu.CompilerParams(needs_layout_passes=False),
        out_type=jax.ShapeDtypeStruct((_NH, _NN), jnp.float32),
        scratch_types=[
            pltpu.VMEM((_ROWS * _NH,), jnp.float32),
            pltpu.VMEM((_CHUNK,), jnp.int32),
            pltpu.VMEM((_NH, _CHUNK), jnp.float32),
        ],
    )
    def body(table_hbm, idx_hbm, out_hbm, tab_v, idx_v, out_v):
        wid = lax.axis_index("s") * 2 + lax.axis_index("c")
        base = wid * _CHUNK
        pltpu.sync_copy(table_hbm, tab_v)
        pltpu.sync_copy(idx_hbm.at[pl.ds(base, _CHUNK)], idx_v)

        @plsc.parallel_loop(0, _CHUNK // _L, unroll=2)
        def step(t):
            off = t * _L
            ivec = idx_v[pl.ds(off, _L)]
            for h in range(_NH):
                out_v[h, pl.ds(off, _L)] = plsc.load_gather(
                    tab_v, [ivec + (h * _ROWS)]
                )
        pltpu.sync_copy(out_v, out_hbm.at[:, pl.ds(base, _CHUNK)])

    return body(table, idx_flat)


def kernel(table, rel_index):
    idx_flat = rel_index[:_N, :_N].reshape(-1)
    out = _sc_gather(table.T.reshape(-1), idx_flat)
    return out.reshape(1, _NH, _N, _N)


# speedup vs baseline: 1.3224x; 1.2888x over previous
import functools

import jax
import jax.numpy as jnp
from jax import lax
from jax.experimental import pallas as pl
from jax.experimental.pallas import tpu as pltpu
from jax.experimental.pallas import tpu_sc as plsc

_N = 256
_NH = 32
_NN = _N * _N
_ROWS = 961
_NW = 32
_CHUNK = _NN // _NW
_L = 16


def _sc_gather(table_t_flat, idx_flat):
    mesh = plsc.VectorSubcoreMesh(core_axis_name="c", subcore_axis_name="s")

    @functools.partial(
        pl.kernel,
        mesh=mesh,
        compiler_params=pltpu.CompilerParams(needs_layout_passes=False),
        out_type=jax.ShapeDtypeStruct((_NH, _NN // 1024, 8, 128), jnp.float32),
        scratch_types=[
            pltpu.VMEM((_ROWS * _NH,), jnp.float32),
            pltpu.VMEM((_CHUNK,), jnp.int32),
            pltpu.VMEM((_NH, 2, 8, 128), jnp.float32),
        ],
    )
    def body(table_hbm, idx_hbm, out_hbm, tab_v, idx_v, out_v):
        wid = lax.axis_index("s") * 2 + lax.axis_index("c")
        base = wid * _CHUNK
        pltpu.sync_copy(table_hbm, tab_v)
        pltpu.sync_copy(idx_hbm.at[pl.ds(base, _CHUNK)], idx_v)

        @plsc.parallel_loop(0, _CHUNK // _L, unroll=2)
        def step(t):
            off = t * _L
            ii = t >> 4
            jb = (t >> 3) & 1
            ji = (t & 7) * _L
            ivec = idx_v[pl.ds(off, _L)]
            for h in range(_NH):
                out_v[h, jb, ii, pl.ds(ji, _L)] = plsc.load_gather(
                    tab_v, [ivec + (h * _ROWS)]
                )

        pltpu.sync_copy(out_v, out_hbm.at[:, pl.ds(wid * 2, 2)])

    return body(table_t_flat, idx_flat)


def kernel(table, rel_index):
    idx_flat = rel_index[:_N, :_N].reshape(-1)
    out = _sc_gather(table.T.reshape(-1), idx_flat)
    out = out.reshape(_NH, _N // 8, _N // 128, 8, 128)
    out = out.transpose(0, 1, 3, 2, 4)
    return out.reshape(1, _NH, _N, _N)


# speedup vs baseline: 1.3591x; 1.0278x over previous
import functools

import jax
import jax.numpy as jnp
from jax import lax
from jax.experimental import pallas as pl
from jax.experimental.pallas import tpu as pltpu
from jax.experimental.pallas import tpu_sc as plsc

_N = 256
_NH = 32
_NN = _N * _N
_ROWS = 961
_NW = 32
_CHUNK = _NN // _NW
_L = 16


def _sc_gather(table_t_flat, idx_flat):
    mesh = plsc.VectorSubcoreMesh(core_axis_name="c", subcore_axis_name="s")

    @functools.partial(
        pl.kernel,
        mesh=mesh,
        compiler_params=pltpu.CompilerParams(needs_layout_passes=False),
        out_type=jax.ShapeDtypeStruct((_NH, _NN // 1024, 8, 128), jnp.float32),
        scratch_types=[
            pltpu.VMEM((_ROWS * _NH,), jnp.float32),
            pltpu.VMEM((_CHUNK,), jnp.int32),
            pltpu.VMEM((_NH, 2, 8, 128), jnp.float32),
            pltpu.SemaphoreType.DMA,
        ],
    )
    def body(table_hbm, idx_hbm, out_hbm, tab_v, idx_v, out_v, sem):
        wid = lax.axis_index("s") * 2 + lax.axis_index("c")
        base = wid * _CHUNK
        pltpu.sync_copy(table_hbm, tab_v)
        pltpu.sync_copy(idx_hbm.at[pl.ds(base, _CHUNK)], idx_v)

        copies = []
        for jb in range(2):

            @plsc.parallel_loop(0, _CHUNK // _L // 2, unroll=2)
            def step(t, _jb=jb):
                ii = t >> 3
                ji = (t & 7) * _L
                off = ii * _N + _jb * 128 + ji
                ivec = idx_v[pl.ds(off, _L)]
                for h in range(_NH):
                    out_v[h, _jb, ii, pl.ds(ji, _L)] = plsc.load_gather(
                        tab_v, [ivec + (h * _ROWS)]
                    )

            copies.append(
                pltpu.async_copy(
                    out_v.at[:, pl.ds(jb, 1)],
                    out_hbm.at[:, pl.ds(wid * 2 + jb, 1)],
                    sem,
                )
            )
        for cp in copies:
            cp.wait()

    return body(table_t_flat, idx_flat)


def kernel(table, rel_index):
    idx_flat = rel_index[:_N, :_N].reshape(-1)
    out = _sc_gather(table.T.reshape(-1), idx_flat)
    out = out.reshape(_NH, _N // 8, _N // 128, 8, 128)
    out = out.transpose(0, 1, 3, 2, 4)
    return out.reshape(1, _NH, _N, _N)


# speedup vs baseline: 1.3654x; 1.0046x over previous
import functools

import jax
import jax.numpy as jnp
from jax import lax
from jax.experimental import pallas as pl
from jax.experimental.pallas import tpu as pltpu
from jax.experimental.pallas import tpu_sc as plsc

_N = 256
_NH = 32
_NN = _N * _N
_ROWS = 961
_NW = 32
_CHUNK = _NN // _NW
_L = 16


def _sc_gather(table_t_flat, idx_flat):
    mesh = plsc.VectorSubcoreMesh(core_axis_name="c", subcore_axis_name="s")

    @functools.partial(
        pl.kernel,
        mesh=mesh,
        compiler_params=pltpu.CompilerParams(needs_layout_passes=False),
        out_type=jax.ShapeDtypeStruct((_NH, _NN // 1024, 8, 128), jnp.float32),
        scratch_types=[
            pltpu.VMEM((_ROWS * _NH,), jnp.float32),
            pltpu.VMEM((_CHUNK,), jnp.int32),
            pltpu.VMEM((_NH, 2, 8, 128), jnp.float32),
            pltpu.SemaphoreType.DMA,
            pltpu.SemaphoreType.DMA,
            pltpu.SemaphoreType.DMA,
            pltpu.SemaphoreType.DMA,
        ],
    )
    def body(
        table_hbm, idx_hbm, out_hbm, tab_v, idx_v, out_v, ts0, ts1, isem, osem
    ):
        wid = lax.axis_index("s") * 2 + lax.axis_index("c")
        base = wid * _CHUNK
        half_tab = _ROWS * _NH // 2

        cp_t0 = pltpu.async_copy(
            table_hbm.at[pl.ds(0, half_tab)], tab_v.at[pl.ds(0, half_tab)], ts0
        )
        cp_i = pltpu.async_copy(idx_hbm.at[pl.ds(base, _CHUNK)], idx_v, isem)
        cp_t1 = pltpu.async_copy(
            table_hbm.at[pl.ds(half_tab, half_tab)],
            tab_v.at[pl.ds(half_tab, half_tab)],
            ts1,
        )
        cp_i.wait()
        cp_t0.wait()

        copies = []
        for hg in range(2):
            if hg == 1:
                cp_t1.wait()
            for jb in range(2):

                @plsc.parallel_loop(0, _CHUNK // _L // 2, unroll=2)
                def step(t, _jb=jb, _hg=hg):
                    ii = t >> 3
                    ji = (t & 7) * _L
                    off = ii * _N + _jb * 128 + ji
                    ivec = idx_v[pl.ds(off, _L)]
                    for h in range(_hg * 16, _hg * 16 + 16):
                        out_v[h, _jb, ii, pl.ds(ji, _L)] = plsc.load_gather(
                            tab_v, [ivec + (h * _ROWS)]
                        )

                copies.append(
                    pltpu.async_copy(
                        out_v.at[pl.ds(hg * 16, 16), pl.ds(jb, 1)],
                        out_hbm.at[pl.ds(hg * 16, 16), pl.ds(wid * 2 + jb, 1)],
                        osem,
                    )
                )
        for cp in copies:
            cp.wait()

    return body(table_t_flat, idx_flat)


def kernel(table, rel_index):
    idx_flat = rel_index[:_N, :_N].reshape(-1)
    out = _sc_gather(table.T.reshape(-1), idx_flat)
    out = out.reshape(_NH, _N // 8, _N // 128, 8, 128)
    out = out.transpose(0, 1, 3, 2, 4)
    return out.reshape(1, _NH, _N, _N)
